# Initial kernel scaffold; baseline (speedup 1.0000x reference)
import jax
import jax.numpy as jnp
from jax.experimental import pallas as pl


def kernel(x):
    raise NotImplementedError("write your pallas kernel here")


# speedup vs baseline: 71.3507x; 71.3507x over previous
import jax
import jax.numpy as jnp
from jax.experimental import pallas as pl
from jax.experimental.pallas import tpu as pltpu


def _twosort_block(x_ref, o_ref):
    v = x_ref[...]
    left = pltpu.roll(v, shift=v.shape[1] - 1, axis=1)
    right = pltpu.roll(v, shift=1, axis=1)
    lane = jax.lax.broadcasted_iota(jnp.int32, v.shape, 1)
    even = (lane % 2) == 0
    partner = jnp.where(even, left, right)
    lo = jnp.minimum(v, partner)
    hi = jnp.maximum(v, partner)
    o_ref[...] = jnp.where(even, lo, hi)


def kernel(x):
    R, C = x.shape
    BR = 256
    return pl.pallas_call(
        _twosort_block,
        out_shape=jax.ShapeDtypeStruct((R, C), x.dtype),
        grid=(R // BR,),
        in_specs=[pl.BlockSpec((BR, C), lambda i: (i, 0))],
        out_specs=pl.BlockSpec((BR, C), lambda i: (i, 0)),
    )(x)
